# baseline (speedup 1.0000x reference)
import jax
import jax.numpy as jnp
from jax.experimental import pallas as pl
from jax.experimental.pallas import tpu as pltpu
import functools

MARGIN_ = 1.0
NLAB_ = 256
KAUG_ = 52


def _split_hi_lo(x):
    hi = x.astype(jnp.bfloat16)
    lo = (x - hi.astype(jnp.float32)).astype(jnp.bfloat16)
    return hi, lo


def _triplet_tc_kernel(n, bm, nblk, e_all, t_all, loss_ref, cnt_ref,
                       v_scr, taf_scr, tcol_scr, valid_scr, bigc_scr):
    i = pl.program_id(0)

    @pl.when(i == 0)
    def _prep():
        a = e_all[...]
        sq_a = jnp.sum(a * a, axis=1, keepdims=True)
        a_hi, a_lo = _split_hi_lo(a)
        s_hi, s_lo = _split_hi_lo(sq_a)
        one = jnp.ones((n, 1), jnp.bfloat16)
        v_scr[:, 0:16] = a_hi
        v_scr[:, 16:32] = a_hi
        v_scr[:, 32:48] = a_lo
        v_scr[:, 48:49] = s_hi
        v_scr[:, 49:50] = s_lo
        v_scr[:, 50:51] = one
        v_scr[:, 51:52] = one
        bigc_scr[0, 0] = 4.0 * jnp.max(sq_a) + 2.0 * MARGIN_ + 1.0

        lab = t_all[...]
        taf_scr[...] = lab.astype(jnp.float32)
        gi = jax.lax.broadcasted_iota(jnp.int32, (NLAB_, n), 0)
        oh = (gi == lab).astype(jnp.float32)
        hist = jnp.sum(oh, axis=1, keepdims=True)

        gf = jax.lax.broadcasted_iota(
            jnp.int32, (NLAB_, 1), 0).astype(jnp.float32)
        tcol_scr[...] = jax.lax.dot_general(
            oh, gf, (((0,), (0,)), ((), ())),
            preferred_element_type=jnp.float32)
        cnt_col = jax.lax.dot_general(
            oh, hist, (((0,), (0,)), ((), ())),
            preferred_element_type=jnp.float32)
        validv = jnp.logical_and(cnt_col >= 2.0, cnt_col <= n - 1.0)
        valid_scr[...] = validv.astype(jnp.float32)

        validh = jnp.logical_and(hist >= 2.0, hist <= n - 1.0)
        nt = jnp.sum(jnp.where(validh, hist, 0.0))
        cnt_ref[0, 0] = nt.astype(jnp.int32)
        loss_ref[0, 0] = 0.0

    e = e_all[pl.ds(i * bm, bm), :]
    sq_r = jnp.sum(e * e, axis=1, keepdims=True)
    e_hi, e_lo = _split_hi_lo(e)
    r_hi, r_lo = _split_hi_lo(sq_r)
    oneb = jnp.ones((bm, 1), jnp.bfloat16)
    u = jnp.concatenate(
        [-2.0 * e_hi, -2.0 * e_lo, -2.0 * e_hi,
         oneb, oneb, r_hi, r_lo], axis=1)

    big_c = bigc_scr[0, 0]
    tf = tcol_scr[pl.ds(i * bm, bm), :]

    ck = 512
    mx = jnp.full((bm, ck), -3.0e38, jnp.float32)
    mn = jnp.full((bm, ck), 3.0e38, jnp.float32)
    for k in range(n // ck):
        vc = v_scr[k * ck:(k + 1) * ck, :]
        dc = jax.lax.dot_general(u, vc, (((1,), (1,)), ((), ())),
                                 preferred_element_type=jnp.float32)
        sc = tf == taf_scr[:, k * ck:(k + 1) * ck]
        zc = jnp.where(sc, dc + big_c, dc)
        mx = jnp.maximum(mx, zc)
        mn = jnp.minimum(mn, zc)

    ap = jnp.max(mx, axis=1, keepdims=True) - big_c
    an = jnp.min(mn, axis=1, keepdims=True)

    vf = valid_scr[pl.ds(i * bm, bm), :]
    losses = vf * jnp.maximum(ap - an + MARGIN_, 0.0)

    loss_ref[0, 0] += jnp.sum(losses)

    @pl.when(i == nblk - 1)
    def _finish():
        denom = jnp.maximum(cnt_ref[0, 0].astype(jnp.float32), 1.0)
        loss_ref[0, 0] = loss_ref[0, 0] / denom


@jax.jit
def kernel(embeddings, target):
    n, d = embeddings.shape
    bm = 2048
    nblk = n // bm
    t_all = target.reshape(1, n)

    loss, cnt = pl.pallas_call(
        functools.partial(_triplet_tc_kernel, n, bm, nblk),
        grid=(nblk,),
        in_specs=[
            pl.BlockSpec((n, d), lambda i: (0, 0)),
            pl.BlockSpec((1, n), lambda i: (0, 0)),
        ],
        out_specs=[
            pl.BlockSpec(memory_space=pltpu.SMEM),
            pl.BlockSpec(memory_space=pltpu.SMEM),
        ],
        out_shape=[
            jax.ShapeDtypeStruct((1, 1), jnp.float32),
            jax.ShapeDtypeStruct((1, 1), jnp.int32),
        ],
        scratch_shapes=[
            pltpu.VMEM((n, KAUG_), jnp.bfloat16),
            pltpu.VMEM((1, n), jnp.float32),
            pltpu.VMEM((n, 1), jnp.float32),
            pltpu.VMEM((n, 1), jnp.float32),
            pltpu.SMEM((1, 1), jnp.float32),
        ],
    )(embeddings, t_all)

    return (loss[0, 0], cnt[0, 0])


# speedup vs baseline: 1.4458x; 1.4458x over previous
import jax
import jax.numpy as jnp
from jax.experimental import pallas as pl
from jax.experimental.pallas import tpu as pltpu
import functools

MARGIN_ = 1.0
NLAB_ = 256


def _triplet_tc_kernel(n, bm, nblk, ck, e_all, t_all, loss_ref, cnt_ref,
                       v_scr, taf_scr, tcol_scr, valid_scr):
    i = pl.program_id(0)

    @pl.when(i == 0)
    def _prep():
        a = e_all[...]
        sq_a = jnp.sum(a * a, axis=1, keepdims=True)
        v_scr[:, 0:16] = a
        v_scr[:, 16:17] = sq_a
        v_scr[:, 17:18] = jnp.ones((n, 1), jnp.float32)

        lab = t_all[...]
        taf_scr[...] = lab.astype(jnp.float32)
        gi = jax.lax.broadcasted_iota(jnp.int32, (NLAB_, n), 0)
        oh = (gi == lab).astype(jnp.float32)
        hist = jnp.sum(oh, axis=1, keepdims=True)

        gf = jax.lax.broadcasted_iota(
            jnp.int32, (NLAB_, 1), 0).astype(jnp.float32)
        tcol_scr[...] = jax.lax.dot_general(
            oh, gf, (((0,), (0,)), ((), ())),
            preferred_element_type=jnp.float32)
        cnt_col = jax.lax.dot_general(
            oh, hist, (((0,), (0,)), ((), ())),
            preferred_element_type=jnp.float32)
        validv = jnp.logical_and(cnt_col >= 2.0, cnt_col <= n - 1.0)
        valid_scr[...] = validv.astype(jnp.float32)

        validh = jnp.logical_and(hist >= 2.0, hist <= n - 1.0)
        nt = jnp.sum(jnp.where(validh, hist, 0.0))
        cnt_ref[0, 0] = nt.astype(jnp.int32)
        loss_ref[0, 0] = 0.0

    e = e_all[pl.ds(i * bm, bm), :]
    sq_r = jnp.sum(e * e, axis=1, keepdims=True)
    u = jnp.concatenate(
        [-2.0 * e, jnp.ones((bm, 1), jnp.float32), sq_r], axis=1)

    big_c = 4.0 * jnp.max(v_scr[:, 16:17]) + 2.0 * MARGIN_ + 1.0
    tf = tcol_scr[pl.ds(i * bm, bm), :]

    mx = jnp.full((bm, ck), -3.0e38, jnp.float32)
    mn = jnp.full((bm, ck), 3.0e38, jnp.float32)
    for k in range(n // ck):
        vc = v_scr[k * ck:(k + 1) * ck, :]
        dc = jax.lax.dot_general(u, vc, (((1,), (1,)), ((), ())),
                                 preferred_element_type=jnp.float32)
        sc = tf == taf_scr[:, k * ck:(k + 1) * ck]
        zc = jnp.where(sc, dc + big_c, dc)
        mx = jnp.maximum(mx, zc)
        mn = jnp.minimum(mn, zc)

    ap = jnp.max(mx, axis=1, keepdims=True) - big_c
    an = jnp.min(mn, axis=1, keepdims=True)

    vf = valid_scr[pl.ds(i * bm, bm), :]
    losses = vf * jnp.maximum(ap - an + MARGIN_, 0.0)

    loss_ref[0, 0] += jnp.sum(losses)

    @pl.when(i == nblk - 1)
    def _finish():
        denom = jnp.maximum(cnt_ref[0, 0].astype(jnp.float32), 1.0)
        loss_ref[0, 0] = loss_ref[0, 0] / denom


@jax.jit
def kernel(embeddings, target):
    n, d = embeddings.shape
    bm = 2048
    ck = 1024
    nblk = n // bm
    t_all = target.reshape(1, n)

    loss, cnt = pl.pallas_call(
        functools.partial(_triplet_tc_kernel, n, bm, nblk, ck),
        grid=(nblk,),
        in_specs=[
            pl.BlockSpec((n, d), lambda i: (0, 0)),
            pl.BlockSpec((1, n), lambda i: (0, 0)),
        ],
        out_specs=[
            pl.BlockSpec(memory_space=pltpu.SMEM),
            pl.BlockSpec(memory_space=pltpu.SMEM),
        ],
        out_shape=[
            jax.ShapeDtypeStruct((1, 1), jnp.float32),
            jax.ShapeDtypeStruct((1, 1), jnp.int32),
        ],
        scratch_shapes=[
            pltpu.VMEM((n, 18), jnp.float32),
            pltpu.VMEM((1, n), jnp.float32),
            pltpu.VMEM((n, 1), jnp.float32),
            pltpu.VMEM((n, 1), jnp.float32),
        ],
    )(embeddings, t_all)

    return (loss[0, 0], cnt[0, 0])
